# Initial kernel scaffold; baseline (speedup 1.0000x reference)
import jax
import jax.numpy as jnp
from jax.experimental import pallas as pl


def kernel(x, edge_index, batch, embed, norm0_gamma, norm0_beta, lin0_W, lin0_b, eps0, mlp0_W1, mlp0_b1, mlp0_bn_gamma, mlp0_bn_beta, mlp0_W2, mlp0_b2, norm1_gamma, norm1_beta, lin1_W, lin1_b, eps1, mlp1_W1, mlp1_b1, mlp1_bn_gamma, mlp1_bn_beta, mlp1_W2, mlp1_b2, readout_W, readout_b):
    raise NotImplementedError("write your pallas kernel here")


# speedup vs baseline: 6.6819x; 6.6819x over previous
import functools

import jax
import jax.numpy as jnp
from jax import lax
from jax.experimental import pallas as pl
from jax.experimental.pallas import tpu as pltpu
from jax.experimental.pallas import tpu_sc as plsc

N = 10000
E = 320000
H = 128
OUT = 128
G = 64
NVOCAB = 100

NC = 2
NS = 16
NW = NC * NS
K = 80
EPT = E // NW
CH = EPT // K
ZR = 640


def _embed_layer_body(x_ref, embed_ref, ng_ref, nb_ref, lw_ref, lb_ref, out_ref):
    xi = x_ref[...]
    onehot = (lax.broadcasted_iota(jnp.int32, (N, NVOCAB), 1) == xi)
    h = jnp.dot(onehot.astype(jnp.float32), embed_ref[...],
                preferred_element_type=jnp.float32)
    mu = jnp.mean(h, axis=0, keepdims=True)
    var = jnp.mean((h - mu) ** 2, axis=0, keepdims=True)
    hn = (h - mu) / jnp.sqrt(var + 1e-5) * ng_ref[...] + nb_ref[...]
    out_ref[...] = jnp.maximum(
        jnp.dot(hn, lw_ref[...], preferred_element_type=jnp.float32)
        + lb_ref[...], 0.0)


def _mid_body(h_ref, p_ref, eps_ref, w1_ref, b1_ref, bg_ref, bb_ref,
              w2_ref, b2_ref, ng_ref, nb_ref, lw_ref, lb_ref, out_ref):
    h = h_ref[...]
    agg = p_ref[0] + p_ref[1]
    z = (1.0 + eps_ref[0, 0]) * h + agg
    z = jnp.dot(z, w1_ref[...], preferred_element_type=jnp.float32) + b1_ref[...]
    mu = jnp.mean(z, axis=0, keepdims=True)
    var = jnp.mean((z - mu) ** 2, axis=0, keepdims=True)
    z = (z - mu) / jnp.sqrt(var + 1e-5) * bg_ref[...] + bb_ref[...]
    z = jnp.maximum(z, 0.0)
    h2 = jnp.dot(z, w2_ref[...], preferred_element_type=jnp.float32) + b2_ref[...]
    mu2 = jnp.mean(h2, axis=0, keepdims=True)
    var2 = jnp.mean((h2 - mu2) ** 2, axis=0, keepdims=True)
    hn = (h2 - mu2) / jnp.sqrt(var2 + 1e-5) * ng_ref[...] + nb_ref[...]
    out_ref[...] = jnp.maximum(
        jnp.dot(hn, lw_ref[...], preferred_element_type=jnp.float32)
        + lb_ref[...], 0.0)


def _final_body(h_ref, p_ref, eps_ref, w1_ref, b1_ref, bg_ref, bb_ref,
                w2_ref, b2_ref, rw_ref, rb_ref, batch_ref, out_ref):
    h = h_ref[...]
    agg = p_ref[0] + p_ref[1]
    z = (1.0 + eps_ref[0, 0]) * h + agg
    z = jnp.dot(z, w1_ref[...], preferred_element_type=jnp.float32) + b1_ref[...]
    mu = jnp.mean(z, axis=0, keepdims=True)
    var = jnp.mean((z - mu) ** 2, axis=0, keepdims=True)
    z = (z - mu) / jnp.sqrt(var + 1e-5) * bg_ref[...] + bb_ref[...]
    z = jnp.maximum(z, 0.0)
    h2 = jnp.dot(z, w2_ref[...], preferred_element_type=jnp.float32) + b2_ref[...]
    r = jnp.dot(h2, rw_ref[...], preferred_element_type=jnp.float32) + rb_ref[...]
    ind = (lax.broadcasted_iota(jnp.int32, (G, N), 0) == batch_ref[...])
    out_ref[...] = jnp.dot(ind.astype(jnp.float32), r,
                           preferred_element_type=jnp.float32)


def _tc_call(body, out_shape):
    return pl.pallas_call(body, out_shape=jax.ShapeDtypeStruct(out_shape,
                                                               jnp.float32))


def _make_sc_agg():
    mesh = plsc.VectorSubcoreMesh(core_axis_name="c", subcore_axis_name="s",
                                  num_cores=NC, num_subcores=NS)

    @functools.partial(
        pl.kernel,
        mesh=mesh,
        out_type=jax.ShapeDtypeStruct((NC, N, H), jnp.float32),
        scratch_types=[
            pltpu.VMEM((CH, K), jnp.int32),
            pltpu.VMEM((CH, K), jnp.int32),
            pltpu.VMEM((K, H), jnp.float32),
            pltpu.VMEM((16, H), jnp.float32),
            pltpu.VMEM_SHARED((N, H), jnp.float32),
            pltpu.SemaphoreType.DMA,
        ],
    )
    def sc_agg(h_hbm, src_hbm, dst_hbm, out_hbm,
               src_v, dst_v, rows_v, zbuf, agg_sh, sem):
        cid = lax.axis_index("c")
        sid = lax.axis_index("s")
        wid = cid * NS + sid

        zero = jnp.zeros((16,), jnp.float32)
        for r in range(16):
            for c in range(H // 16):
                zbuf[r, pl.ds(c * 16, 16)] = zero

        def zbody(j, carry):
            off = pl.multiple_of(jnp.minimum(sid * ZR + j * 16, N - 16), 16)
            pltpu.sync_copy(zbuf, agg_sh.at[pl.ds(off, 16)])
            return carry
        lax.fori_loop(0, ZR // 16, zbody, 0)
        plsc.subcore_barrier()

        pltpu.sync_copy(src_hbm.at[wid], src_v)
        pltpu.sync_copy(dst_hbm.at[wid], dst_v)

        def ebody(j, carry):
            pltpu.async_copy(h_hbm.at[src_v.at[j]], rows_v, sem).wait()
            pltpu.sync_copy(rows_v, agg_sh.at[dst_v.at[j]], add=True)
            return carry
        lax.fori_loop(0, CH, ebody, 0)
        plsc.subcore_barrier()

        wb = pl.multiple_of(jnp.minimum(sid * ZR, N - ZR), 16)
        pltpu.sync_copy(agg_sh.at[pl.ds(wb, ZR)],
                        out_hbm.at[cid, pl.ds(wb, ZR)])

    return sc_agg


_SC_AGG_CACHE = []


def _sc_agg(h, src, dst):
    if not _SC_AGG_CACHE:
        _SC_AGG_CACHE.append(_make_sc_agg())
    return _SC_AGG_CACHE[0](h, src, dst)


def kernel(x, edge_index, batch, embed,
           norm0_gamma, norm0_beta, lin0_W, lin0_b, eps0,
           mlp0_W1, mlp0_b1, mlp0_bn_gamma, mlp0_bn_beta, mlp0_W2, mlp0_b2,
           norm1_gamma, norm1_beta, lin1_W, lin1_b, eps1,
           mlp1_W1, mlp1_b1, mlp1_bn_gamma, mlp1_bn_beta, mlp1_W2, mlp1_b2,
           readout_W, readout_b):
    src = edge_index[0].reshape(NW, CH, K)
    dst = edge_index[1].reshape(NW, CH, K)
    x2 = x.reshape(N, 1)
    batch2 = batch.reshape(1, N)
    row = lambda v: v.reshape(1, -1)

    h1 = _tc_call(_embed_layer_body, (N, H))(
        x2, embed, row(norm0_gamma), row(norm0_beta), lin0_W, row(lin0_b))

    p0 = _sc_agg(h1, src, dst)

    h3 = _tc_call(_mid_body, (N, H))(
        h1, p0, eps0.reshape(1, 1), mlp0_W1, row(mlp0_b1),
        row(mlp0_bn_gamma), row(mlp0_bn_beta), mlp0_W2, row(mlp0_b2),
        row(norm1_gamma), row(norm1_beta), lin1_W, row(lin1_b))

    p1 = _sc_agg(h3, src, dst)

    out = _tc_call(_final_body, (G, OUT))(
        h3, p1, eps1.reshape(1, 1), mlp1_W1, row(mlp1_b1),
        row(mlp1_bn_gamma), row(mlp1_bn_beta), mlp1_W2, row(mlp1_b2),
        readout_W, row(readout_b), batch2)

    return out


# speedup vs baseline: 10.5067x; 1.5724x over previous
import functools

import jax
import jax.numpy as jnp
from jax import lax
from jax.experimental import pallas as pl
from jax.experimental.pallas import tpu as pltpu
from jax.experimental.pallas import tpu_sc as plsc

N = 10000
E = 320000
H = 128
OUT = 128
G = 64
NVOCAB = 100

NC = 2
NS = 16
NW = NC * NS
K = 80
EPT = E // NW
CH = EPT // K
ZR = 640


def _embed_layer_body(x_ref, embed_ref, ng_ref, nb_ref, lw_ref, lb_ref, out_ref):
    xi = x_ref[...]
    onehot = (lax.broadcasted_iota(jnp.int32, (N, NVOCAB), 1) == xi)
    h = jnp.dot(onehot.astype(jnp.float32), embed_ref[...],
                preferred_element_type=jnp.float32)
    mu = jnp.mean(h, axis=0, keepdims=True)
    var = jnp.mean((h - mu) ** 2, axis=0, keepdims=True)
    hn = (h - mu) / jnp.sqrt(var + 1e-5) * ng_ref[...] + nb_ref[...]
    out_ref[...] = jnp.maximum(
        jnp.dot(hn, lw_ref[...], preferred_element_type=jnp.float32)
        + lb_ref[...], 0.0)


def _mid_body(h_ref, p_ref, eps_ref, w1_ref, b1_ref, bg_ref, bb_ref,
              w2_ref, b2_ref, ng_ref, nb_ref, lw_ref, lb_ref, out_ref):
    h = h_ref[...]
    agg = p_ref[0] + p_ref[1]
    z = (1.0 + eps_ref[0, 0]) * h + agg
    z = jnp.dot(z, w1_ref[...], preferred_element_type=jnp.float32) + b1_ref[...]
    mu = jnp.mean(z, axis=0, keepdims=True)
    var = jnp.mean((z - mu) ** 2, axis=0, keepdims=True)
    z = (z - mu) / jnp.sqrt(var + 1e-5) * bg_ref[...] + bb_ref[...]
    z = jnp.maximum(z, 0.0)
    h2 = jnp.dot(z, w2_ref[...], preferred_element_type=jnp.float32) + b2_ref[...]
    mu2 = jnp.mean(h2, axis=0, keepdims=True)
    var2 = jnp.mean((h2 - mu2) ** 2, axis=0, keepdims=True)
    hn = (h2 - mu2) / jnp.sqrt(var2 + 1e-5) * ng_ref[...] + nb_ref[...]
    out_ref[...] = jnp.maximum(
        jnp.dot(hn, lw_ref[...], preferred_element_type=jnp.float32)
        + lb_ref[...], 0.0)


def _final_body(h_ref, p_ref, eps_ref, w1_ref, b1_ref, bg_ref, bb_ref,
                w2_ref, b2_ref, rw_ref, rb_ref, batch_ref, out_ref):
    h = h_ref[...]
    agg = p_ref[0] + p_ref[1]
    z = (1.0 + eps_ref[0, 0]) * h + agg
    z = jnp.dot(z, w1_ref[...], preferred_element_type=jnp.float32) + b1_ref[...]
    mu = jnp.mean(z, axis=0, keepdims=True)
    var = jnp.mean((z - mu) ** 2, axis=0, keepdims=True)
    z = (z - mu) / jnp.sqrt(var + 1e-5) * bg_ref[...] + bb_ref[...]
    z = jnp.maximum(z, 0.0)
    h2 = jnp.dot(z, w2_ref[...], preferred_element_type=jnp.float32) + b2_ref[...]
    r = jnp.dot(h2, rw_ref[...], preferred_element_type=jnp.float32) + rb_ref[...]
    ind = (lax.broadcasted_iota(jnp.int32, (G, N), 0) == batch_ref[...])
    out_ref[...] = jnp.dot(ind.astype(jnp.float32), r,
                           preferred_element_type=jnp.float32)


def _tc_call(body, out_shape):
    return pl.pallas_call(body, out_shape=jax.ShapeDtypeStruct(out_shape,
                                                               jnp.float32))


def _make_sc_agg():
    mesh = plsc.VectorSubcoreMesh(core_axis_name="c", subcore_axis_name="s",
                                  num_cores=NC, num_subcores=NS)

    @functools.partial(
        pl.kernel,
        mesh=mesh,
        out_type=jax.ShapeDtypeStruct((NC, N, H), jnp.float32),
        scratch_types=[
            pltpu.VMEM((EPT,), jnp.int32),
            pltpu.VMEM((CH, K), jnp.int32),
            pltpu.VMEM((K, H), jnp.float32),
            pltpu.VMEM((K, H), jnp.float32),
            pltpu.VMEM((16, H), jnp.float32),
            pltpu.VMEM_SHARED((N, H), jnp.float32),
            pltpu.SemaphoreType.DMA,
            pltpu.SemaphoreType.DMA,
        ],
    )
    def sc_agg(h_hbm, src_hbm, dst_hbm, out_hbm,
               src_v, dst_v, rows0, rows1, zbuf, agg_sh, sem0, sem1):
        cid = lax.axis_index("c")
        sid = lax.axis_index("s")
        wid = cid * NS + sid

        zero = jnp.zeros((16,), jnp.float32)
        for r in range(16):
            for c in range(H // 16):
                zbuf[r, pl.ds(c * 16, 16)] = zero

        def zbody(j, carry):
            off = pl.multiple_of(jnp.minimum(sid * ZR + j * 16, N - 16), 16)
            pltpu.sync_copy(zbuf, agg_sh.at[pl.ds(off, 16)])
            return carry
        lax.fori_loop(0, ZR // 16, zbody, 0)
        plsc.subcore_barrier()

        pltpu.sync_copy(src_hbm.at[wid], src_v)
        pltpu.sync_copy(dst_hbm.at[wid], dst_v)

        def sidx(j):
            return src_v.at[pl.ds(pl.multiple_of(j * K, 16), K)]

        pltpu.async_copy(h_hbm.at[sidx(0)], rows0, sem0)

        def gbody(g, carry):
            j = 2 * g
            d1 = pltpu.async_copy(h_hbm.at[sidx(j + 1)], rows1, sem1)
            pltpu.make_async_copy(h_hbm.at[sidx(j)], rows0, sem0).wait()
            pltpu.sync_copy(rows0, agg_sh.at[dst_v.at[j]], add=True)
            pltpu.async_copy(h_hbm.at[sidx(j + 2)], rows0, sem0)
            d1.wait()
            pltpu.sync_copy(rows1, agg_sh.at[dst_v.at[j + 1]], add=True)
            return carry
        lax.fori_loop(0, (CH - 1) // 2, gbody, 0)
        pltpu.make_async_copy(h_hbm.at[sidx(CH - 1)], rows0, sem0).wait()
        pltpu.sync_copy(rows0, agg_sh.at[dst_v.at[CH - 1]], add=True)
        plsc.subcore_barrier()

        wb = pl.multiple_of(jnp.minimum(sid * ZR, N - ZR), 16)
        pltpu.sync_copy(agg_sh.at[pl.ds(wb, ZR)],
                        out_hbm.at[cid, pl.ds(wb, ZR)])

    return sc_agg


_SC_AGG_CACHE = []


def _sc_agg(h, src, dst):
    if not _SC_AGG_CACHE:
        _SC_AGG_CACHE.append(_make_sc_agg())
    return _SC_AGG_CACHE[0](h, src, dst)


def kernel(x, edge_index, batch, embed,
           norm0_gamma, norm0_beta, lin0_W, lin0_b, eps0,
           mlp0_W1, mlp0_b1, mlp0_bn_gamma, mlp0_bn_beta, mlp0_W2, mlp0_b2,
           norm1_gamma, norm1_beta, lin1_W, lin1_b, eps1,
           mlp1_W1, mlp1_b1, mlp1_bn_gamma, mlp1_bn_beta, mlp1_W2, mlp1_b2,
           readout_W, readout_b):
    src = edge_index[0].reshape(NW, EPT)
    dst = edge_index[1].reshape(NW, CH, K)
    x2 = x.reshape(N, 1)
    batch2 = batch.reshape(1, N)
    row = lambda v: v.reshape(1, -1)

    h1 = _tc_call(_embed_layer_body, (N, H))(
        x2, embed, row(norm0_gamma), row(norm0_beta), lin0_W, row(lin0_b))

    p0 = _sc_agg(h1, src, dst)

    h3 = _tc_call(_mid_body, (N, H))(
        h1, p0, eps0.reshape(1, 1), mlp0_W1, row(mlp0_b1),
        row(mlp0_bn_gamma), row(mlp0_bn_beta), mlp0_W2, row(mlp0_b2),
        row(norm1_gamma), row(norm1_beta), lin1_W, row(lin1_b))

    p1 = _sc_agg(h3, src, dst)

    out = _tc_call(_final_body, (G, OUT))(
        h3, p1, eps1.reshape(1, 1), mlp1_W1, row(mlp1_b1),
        row(mlp1_bn_gamma), row(mlp1_bn_beta), mlp1_W2, row(mlp1_b2),
        readout_W, row(readout_b), batch2)

    return out


# speedup vs baseline: 10.8062x; 1.0285x over previous
import functools

import jax
import jax.numpy as jnp
from jax import lax
from jax.experimental import pallas as pl
from jax.experimental.pallas import tpu as pltpu
from jax.experimental.pallas import tpu_sc as plsc

N = 10000
E = 320000
H = 128
OUT = 128
G = 64
NVOCAB = 100

NC = 2
NS = 16
NW = NC * NS
K = 80
EPT = E // NW
CH = EPT // K
ZR = 640


def _embed_layer_body(x_ref, embed_ref, ng_ref, nb_ref, lw_ref, lb_ref, out_ref):
    xi = x_ref[...]
    onehot = (lax.broadcasted_iota(jnp.int32, (N, NVOCAB), 1) == xi)
    h = jnp.dot(onehot.astype(jnp.float32), embed_ref[...],
                preferred_element_type=jnp.float32)
    mu = jnp.mean(h, axis=0, keepdims=True)
    var = jnp.mean((h - mu) ** 2, axis=0, keepdims=True)
    hn = (h - mu) / jnp.sqrt(var + 1e-5) * ng_ref[...] + nb_ref[...]
    out_ref[...] = jnp.maximum(
        jnp.dot(hn, lw_ref[...], preferred_element_type=jnp.float32)
        + lb_ref[...], 0.0)


def _mid_body(h_ref, p_ref, eps_ref, w1_ref, b1_ref, bg_ref, bb_ref,
              w2_ref, b2_ref, ng_ref, nb_ref, lw_ref, lb_ref, out_ref):
    h = h_ref[...]
    agg = p_ref[0] + p_ref[1]
    z = (1.0 + eps_ref[0, 0]) * h + agg
    z = jnp.dot(z, w1_ref[...], preferred_element_type=jnp.float32) + b1_ref[...]
    mu = jnp.mean(z, axis=0, keepdims=True)
    var = jnp.mean((z - mu) ** 2, axis=0, keepdims=True)
    z = (z - mu) / jnp.sqrt(var + 1e-5) * bg_ref[...] + bb_ref[...]
    z = jnp.maximum(z, 0.0)
    h2 = jnp.dot(z, w2_ref[...], preferred_element_type=jnp.float32) + b2_ref[...]
    mu2 = jnp.mean(h2, axis=0, keepdims=True)
    var2 = jnp.mean((h2 - mu2) ** 2, axis=0, keepdims=True)
    hn = (h2 - mu2) / jnp.sqrt(var2 + 1e-5) * ng_ref[...] + nb_ref[...]
    out_ref[...] = jnp.maximum(
        jnp.dot(hn, lw_ref[...], preferred_element_type=jnp.float32)
        + lb_ref[...], 0.0)


def _final_body(h_ref, p_ref, eps_ref, w1_ref, b1_ref, bg_ref, bb_ref,
                w2_ref, b2_ref, rw_ref, rb_ref, batch_ref, out_ref):
    h = h_ref[...]
    agg = p_ref[0] + p_ref[1]
    z = (1.0 + eps_ref[0, 0]) * h + agg
    z = jnp.dot(z, w1_ref[...], preferred_element_type=jnp.float32) + b1_ref[...]
    mu = jnp.mean(z, axis=0, keepdims=True)
    var = jnp.mean((z - mu) ** 2, axis=0, keepdims=True)
    z = (z - mu) / jnp.sqrt(var + 1e-5) * bg_ref[...] + bb_ref[...]
    z = jnp.maximum(z, 0.0)
    h2 = jnp.dot(z, w2_ref[...], preferred_element_type=jnp.float32) + b2_ref[...]
    r = jnp.dot(h2, rw_ref[...], preferred_element_type=jnp.float32) + rb_ref[...]
    ind = (lax.broadcasted_iota(jnp.int32, (G, N), 0) == batch_ref[...])
    out_ref[...] = jnp.dot(ind.astype(jnp.float32), r,
                           preferred_element_type=jnp.float32)


def _tc_call(body, out_shape):
    return pl.pallas_call(body, out_shape=jax.ShapeDtypeStruct(out_shape,
                                                               jnp.float32))


def _make_sc_agg():
    mesh = plsc.VectorSubcoreMesh(core_axis_name="c", subcore_axis_name="s",
                                  num_cores=NC, num_subcores=NS)

    @functools.partial(
        pl.kernel,
        mesh=mesh,
        out_type=jax.ShapeDtypeStruct((NC, N, H), jnp.float32),
        scratch_types=[
            pltpu.VMEM((EPT,), jnp.int32),
            pltpu.VMEM((CH, K), jnp.int32),
            pltpu.VMEM((K, H), jnp.float32),
            pltpu.VMEM((K, H), jnp.float32),
            pltpu.VMEM((16, H), jnp.float32),
            pltpu.VMEM_SHARED((N, H), jnp.float32),
            pltpu.SemaphoreType.DMA,
            pltpu.SemaphoreType.DMA,
            pltpu.SemaphoreType.DMA,
        ],
    )
    def sc_agg(h_hbm, src_hbm, dst_hbm, out_hbm,
               src_v, dst_v, rows0, rows1, zbuf, agg_sh, sem0, sem1, semz):
        cid = lax.axis_index("c")
        sid = lax.axis_index("s")
        wid = cid * NS + sid

        zero = jnp.zeros((16,), jnp.float32)
        for r in range(16):
            for c in range(H // 16):
                zbuf[r, pl.ds(c * 16, 16)] = zero

        def zbody(j, carry):
            off = pl.multiple_of(jnp.minimum(sid * ZR + j * 16, N - 16), 16)
            pltpu.async_copy(zbuf, agg_sh.at[pl.ds(off, 16)], semz)
            return carry
        lax.fori_loop(0, ZR // 16, zbody, 0)

        pltpu.sync_copy(src_hbm.at[wid], src_v)
        pltpu.sync_copy(dst_hbm.at[wid], dst_v)

        def sidx(j):
            return src_v.at[pl.ds(pl.multiple_of(j * K, 16), K)]

        pltpu.async_copy(h_hbm.at[sidx(0)], rows0, sem0)

        def zdrain(j, carry):
            pltpu.make_async_copy(zbuf, agg_sh.at[pl.ds(0, 16)], semz).wait()
            return carry
        lax.fori_loop(0, ZR // 16, zdrain, 0)
        plsc.subcore_barrier()

        def gbody(g, carry):
            j = 2 * g
            d1 = pltpu.async_copy(h_hbm.at[sidx(j + 1)], rows1, sem1)
            pltpu.make_async_copy(h_hbm.at[sidx(j)], rows0, sem0).wait()
            pltpu.sync_copy(rows0, agg_sh.at[dst_v.at[j]], add=True)
            pltpu.async_copy(h_hbm.at[sidx(j + 2)], rows0, sem0)
            d1.wait()
            pltpu.sync_copy(rows1, agg_sh.at[dst_v.at[j + 1]], add=True)
            return carry
        lax.fori_loop(0, (CH - 1) // 2, gbody, 0)
        pltpu.make_async_copy(h_hbm.at[sidx(CH - 1)], rows0, sem0).wait()
        pltpu.sync_copy(rows0, agg_sh.at[dst_v.at[CH - 1]], add=True)
        plsc.subcore_barrier()

        wb = pl.multiple_of(jnp.minimum(sid * ZR, N - ZR), 16)
        pltpu.sync_copy(agg_sh.at[pl.ds(wb, ZR)],
                        out_hbm.at[cid, pl.ds(wb, ZR)])

    return sc_agg


_SC_AGG_CACHE = []


def _sc_agg(h, src, dst):
    if not _SC_AGG_CACHE:
        _SC_AGG_CACHE.append(_make_sc_agg())
    return _SC_AGG_CACHE[0](h, src, dst)


def kernel(x, edge_index, batch, embed,
           norm0_gamma, norm0_beta, lin0_W, lin0_b, eps0,
           mlp0_W1, mlp0_b1, mlp0_bn_gamma, mlp0_bn_beta, mlp0_W2, mlp0_b2,
           norm1_gamma, norm1_beta, lin1_W, lin1_b, eps1,
           mlp1_W1, mlp1_b1, mlp1_bn_gamma, mlp1_bn_beta, mlp1_W2, mlp1_b2,
           readout_W, readout_b):
    src = edge_index[0].reshape(NW, EPT)
    dst = edge_index[1].reshape(NW, CH, K)
    x2 = x.reshape(N, 1)
    batch2 = batch.reshape(1, N)
    row = lambda v: v.reshape(1, -1)

    h1 = _tc_call(_embed_layer_body, (N, H))(
        x2, embed, row(norm0_gamma), row(norm0_beta), lin0_W, row(lin0_b))

    p0 = _sc_agg(h1, src, dst)

    h3 = _tc_call(_mid_body, (N, H))(
        h1, p0, eps0.reshape(1, 1), mlp0_W1, row(mlp0_b1),
        row(mlp0_bn_gamma), row(mlp0_bn_beta), mlp0_W2, row(mlp0_b2),
        row(norm1_gamma), row(norm1_beta), lin1_W, row(lin1_b))

    p1 = _sc_agg(h3, src, dst)

    out = _tc_call(_final_body, (G, OUT))(
        h3, p1, eps1.reshape(1, 1), mlp1_W1, row(mlp1_b1),
        row(mlp1_bn_gamma), row(mlp1_bn_beta), mlp1_W2, row(mlp1_b2),
        readout_W, row(readout_b), batch2)

    return out


# speedup vs baseline: 11.9402x; 1.1049x over previous
import functools

import jax
import jax.numpy as jnp
from jax import lax
from jax.experimental import pallas as pl
from jax.experimental.pallas import tpu as pltpu
from jax.experimental.pallas import tpu_sc as plsc

N = 10000
E = 320000
H = 128
OUT = 128
G = 64
NVOCAB = 100

NC = 2
NS = 16
NW = NC * NS
K = 128
NCH = E // K
CWB = NCH // NW
NXW = NCH - CWB * NW
RS = 4
ZR = 640


def _embed_layer_body(x_ref, embed_ref, ng_ref, nb_ref, lw_ref, lb_ref, out_ref):
    xi = x_ref[...]
    onehot = (lax.broadcasted_iota(jnp.int32, (N, NVOCAB), 1) == xi)
    h = jnp.dot(onehot.astype(jnp.float32), embed_ref[...],
                preferred_element_type=jnp.float32)
    mu = jnp.mean(h, axis=0, keepdims=True)
    var = jnp.mean((h - mu) ** 2, axis=0, keepdims=True)
    hn = (h - mu) / jnp.sqrt(var + 1e-5) * ng_ref[...] + nb_ref[...]
    out_ref[...] = jnp.maximum(
        jnp.dot(hn, lw_ref[...], preferred_element_type=jnp.float32)
        + lb_ref[...], 0.0)


def _mid_body(h_ref, p_ref, eps_ref, w1_ref, b1_ref, bg_ref, bb_ref,
              w2_ref, b2_ref, ng_ref, nb_ref, lw_ref, lb_ref, out_ref):
    h = h_ref[...]
    agg = p_ref[0] + p_ref[1]
    z = (1.0 + eps_ref[0, 0]) * h + agg
    z = jnp.dot(z, w1_ref[...], preferred_element_type=jnp.float32) + b1_ref[...]
    mu = jnp.mean(z, axis=0, keepdims=True)
    var = jnp.mean((z - mu) ** 2, axis=0, keepdims=True)
    z = (z - mu) / jnp.sqrt(var + 1e-5) * bg_ref[...] + bb_ref[...]
    z = jnp.maximum(z, 0.0)
    h2 = jnp.dot(z, w2_ref[...], preferred_element_type=jnp.float32) + b2_ref[...]
    mu2 = jnp.mean(h2, axis=0, keepdims=True)
    var2 = jnp.mean((h2 - mu2) ** 2, axis=0, keepdims=True)
    hn = (h2 - mu2) / jnp.sqrt(var2 + 1e-5) * ng_ref[...] + nb_ref[...]
    out_ref[...] = jnp.maximum(
        jnp.dot(hn, lw_ref[...], preferred_element_type=jnp.float32)
        + lb_ref[...], 0.0)


def _final_body(h_ref, p_ref, eps_ref, w1_ref, b1_ref, bg_ref, bb_ref,
                w2_ref, b2_ref, rw_ref, rb_ref, batch_ref, out_ref):
    h = h_ref[...]
    agg = p_ref[0] + p_ref[1]
    z = (1.0 + eps_ref[0, 0]) * h + agg
    z = jnp.dot(z, w1_ref[...], preferred_element_type=jnp.float32) + b1_ref[...]
    mu = jnp.mean(z, axis=0, keepdims=True)
    var = jnp.mean((z - mu) ** 2, axis=0, keepdims=True)
    z = (z - mu) / jnp.sqrt(var + 1e-5) * bg_ref[...] + bb_ref[...]
    z = jnp.maximum(z, 0.0)
    h2 = jnp.dot(z, w2_ref[...], preferred_element_type=jnp.float32) + b2_ref[...]
    r = jnp.dot(h2, rw_ref[...], preferred_element_type=jnp.float32) + rb_ref[...]
    ind = (lax.broadcasted_iota(jnp.int32, (G, N), 0) == batch_ref[...])
    out_ref[...] = jnp.dot(ind.astype(jnp.float32), r,
                           preferred_element_type=jnp.float32)


def _tc_call(body, out_shape):
    return pl.pallas_call(body, out_shape=jax.ShapeDtypeStruct(out_shape,
                                                               jnp.float32))


def _make_sc_agg():
    mesh = plsc.VectorSubcoreMesh(core_axis_name="c", subcore_axis_name="s",
                                  num_cores=NC, num_subcores=NS)

    @functools.partial(
        pl.kernel,
        mesh=mesh,
        out_type=jax.ShapeDtypeStruct((NC, N, H), jnp.float32),
        scratch_types=[
            pltpu.VMEM((RS, K), jnp.int32),
            pltpu.VMEM((RS, K), jnp.int32),
            pltpu.VMEM((K, H), jnp.float32),
            pltpu.VMEM((K, H), jnp.float32),
            pltpu.VMEM((16, H), jnp.float32),
            pltpu.VMEM_SHARED((N, H), jnp.float32),
            pltpu.SemaphoreType.DMA,
            pltpu.SemaphoreType.DMA,
            pltpu.SemaphoreType.DMA,
            pltpu.SemaphoreType.DMA,
            pltpu.SemaphoreType.DMA,
        ],
    )
    def sc_agg(h_hbm, src_hbm, dst_hbm, out_hbm,
               sring, dring, rows0, rows1, zbuf, agg_sh,
               sem0, sem1, semz, semi0, semi1):
        cid = lax.axis_index("c")
        sid = lax.axis_index("s")
        wid = cid * NS + sid
        cw = CWB + jnp.where(wid >= NW - NXW, 1, 0)
        sw = CWB * wid + jnp.maximum(wid - (NW - NXW), 0)

        zero = jnp.zeros((16,), jnp.float32)
        for r in range(16):
            for c in range(H // 16):
                zbuf[r, pl.ds(c * 16, 16)] = zero

        def zbody(j, carry):
            off = pl.multiple_of(jnp.minimum(sid * ZR + j * 16, N - 16), 16)
            pltpu.async_copy(zbuf, agg_sh.at[pl.ds(off, 16)], semz)
            return carry
        lax.fori_loop(0, ZR // 16, zbody, 0)

        def stage(t, slot, sem):
            off = pl.multiple_of((sw + t) * K, 128)
            pltpu.async_copy(src_hbm.at[pl.ds(off, K)], sring.at[slot], sem)
            pltpu.async_copy(dst_hbm.at[pl.ds(off, K)], dring.at[slot], sem)

        def stage_wait(sem):
            pltpu.make_async_copy(src_hbm.at[pl.ds(0, K)],
                                  sring.at[0], sem).wait()
            pltpu.make_async_copy(dst_hbm.at[pl.ds(0, K)],
                                  dring.at[0], sem).wait()

        def gwait(buf, sem):
            pltpu.make_async_copy(h_hbm.at[sring.at[0]], buf, sem).wait()

        stage(0, 0, semi0)
        stage_wait(semi0)
        stage(1, 1, semi1)
        stage(2, 2, semi0)
        pltpu.async_copy(h_hbm.at[sring.at[0]], rows0, sem0)

        def zdrain(j, carry):
            pltpu.make_async_copy(zbuf, agg_sh.at[pl.ds(0, 16)], semz).wait()
            return carry
        lax.fori_loop(0, ZR // 16, zdrain, 0)
        plsc.subcore_barrier()

        def gbody(g, carry):
            t = 2 * g
            s1 = lax.rem(t + 1, RS)
            s2 = lax.rem(t + 2, RS)
            s3 = lax.rem(t + 3, RS)
            s0 = lax.rem(t, RS)
            stage_wait(semi1)
            pltpu.async_copy(h_hbm.at[sring.at[s1]], rows1, sem1)

            @pl.when(t + 3 < cw)
            def _():
                stage(t + 3, s3, semi1)

            gwait(rows0, sem0)
            pltpu.sync_copy(rows0, agg_sh.at[dring.at[s0]], add=True)

            @pl.when(t + 2 < cw)
            def _():
                stage_wait(semi0)
                pltpu.async_copy(h_hbm.at[sring.at[s2]], rows0, sem0)

            @pl.when(t + 4 < cw)
            def _():
                stage(t + 4, s0, semi0)

            gwait(rows1, sem1)
            pltpu.sync_copy(rows1, agg_sh.at[dring.at[s1]], add=True)
            return carry
        lax.fori_loop(0, CWB // 2, gbody, 0)

        @pl.when(cw > CWB)
        def _():
            gwait(rows0, sem0)
            pltpu.sync_copy(rows0,
                            agg_sh.at[dring.at[lax.rem(CWB, RS)]], add=True)
        plsc.subcore_barrier()

        wb = pl.multiple_of(jnp.minimum(sid * ZR, N - ZR), 16)
        pltpu.sync_copy(agg_sh.at[pl.ds(wb, ZR)],
                        out_hbm.at[cid, pl.ds(wb, ZR)])

    return sc_agg


_SC_AGG_CACHE = []


def _sc_agg(h, src, dst):
    if not _SC_AGG_CACHE:
        _SC_AGG_CACHE.append(_make_sc_agg())
    return _SC_AGG_CACHE[0](h, src, dst)


def kernel(x, edge_index, batch, embed,
           norm0_gamma, norm0_beta, lin0_W, lin0_b, eps0,
           mlp0_W1, mlp0_b1, mlp0_bn_gamma, mlp0_bn_beta, mlp0_W2, mlp0_b2,
           norm1_gamma, norm1_beta, lin1_W, lin1_b, eps1,
           mlp1_W1, mlp1_b1, mlp1_bn_gamma, mlp1_bn_beta, mlp1_W2, mlp1_b2,
           readout_W, readout_b):
    src = edge_index[0]
    dst = edge_index[1]
    x2 = x.reshape(N, 1)
    batch2 = batch.reshape(1, N)
    row = lambda v: v.reshape(1, -1)

    h1 = _tc_call(_embed_layer_body, (N, H))(
        x2, embed, row(norm0_gamma), row(norm0_beta), lin0_W, row(lin0_b))

    p0 = _sc_agg(h1, src, dst)

    h3 = _tc_call(_mid_body, (N, H))(
        h1, p0, eps0.reshape(1, 1), mlp0_W1, row(mlp0_b1),
        row(mlp0_bn_gamma), row(mlp0_bn_beta), mlp0_W2, row(mlp0_b2),
        row(norm1_gamma), row(norm1_beta), lin1_W, row(lin1_b))

    p1 = _sc_agg(h3, src, dst)

    out = _tc_call(_final_body, (G, OUT))(
        h3, p1, eps1.reshape(1, 1), mlp1_W1, row(mlp1_b1),
        row(mlp1_bn_gamma), row(mlp1_bn_beta), mlp1_W2, row(mlp1_b2),
        readout_W, row(readout_b), batch2)

    return out


# speedup vs baseline: 12.5557x; 1.0515x over previous
import functools

import jax
import jax.numpy as jnp
from jax import lax
from jax.experimental import pallas as pl
from jax.experimental.pallas import tpu as pltpu
from jax.experimental.pallas import tpu_sc as plsc

N = 10000
E = 320000
H = 128
OUT = 128
G = 64
NVOCAB = 100

NC = 2
NS = 16
NW = NC * NS
K = 128
NCH = E // K
CWB = NCH // NW
NXW = NCH - CWB * NW
RS = 4
ZR = 640


def _embed_layer_body(x_ref, embed_ref, ng_ref, nb_ref, lw_ref, lb_ref, out_ref):
    xi = x_ref[...]
    onehot = (lax.broadcasted_iota(jnp.int32, (N, NVOCAB), 1) == xi)
    h = jnp.dot(onehot.astype(jnp.float32), embed_ref[...],
                preferred_element_type=jnp.float32)
    mu = jnp.mean(h, axis=0, keepdims=True)
    var = jnp.mean((h - mu) ** 2, axis=0, keepdims=True)
    hn = (h - mu) / jnp.sqrt(var + 1e-5) * ng_ref[...] + nb_ref[...]
    out_ref[...] = jnp.maximum(
        jnp.dot(hn, lw_ref[...], preferred_element_type=jnp.float32)
        + lb_ref[...], 0.0)


def _mid_body(h_ref, p_ref, eps_ref, w1_ref, b1_ref, bg_ref, bb_ref,
              w2_ref, b2_ref, ng_ref, nb_ref, lw_ref, lb_ref, out_ref):
    h = h_ref[...]
    agg = p_ref[0] + p_ref[1]
    z = (1.0 + eps_ref[0, 0]) * h + agg
    z = jnp.dot(z, w1_ref[...], preferred_element_type=jnp.float32) + b1_ref[...]
    mu = jnp.mean(z, axis=0, keepdims=True)
    var = jnp.mean((z - mu) ** 2, axis=0, keepdims=True)
    z = (z - mu) / jnp.sqrt(var + 1e-5) * bg_ref[...] + bb_ref[...]
    z = jnp.maximum(z, 0.0)
    h2 = jnp.dot(z, w2_ref[...], preferred_element_type=jnp.float32) + b2_ref[...]
    mu2 = jnp.mean(h2, axis=0, keepdims=True)
    var2 = jnp.mean((h2 - mu2) ** 2, axis=0, keepdims=True)
    hn = (h2 - mu2) / jnp.sqrt(var2 + 1e-5) * ng_ref[...] + nb_ref[...]
    out_ref[...] = jnp.maximum(
        jnp.dot(hn, lw_ref[...], preferred_element_type=jnp.float32)
        + lb_ref[...], 0.0)


def _final_body(h_ref, p_ref, eps_ref, w1_ref, b1_ref, bg_ref, bb_ref,
                w2_ref, b2_ref, rw_ref, rb_ref, batch_ref, out_ref):
    h = h_ref[...]
    agg = p_ref[0] + p_ref[1]
    z = (1.0 + eps_ref[0, 0]) * h + agg
    z = jnp.dot(z, w1_ref[...], preferred_element_type=jnp.float32) + b1_ref[...]
    mu = jnp.mean(z, axis=0, keepdims=True)
    var = jnp.mean((z - mu) ** 2, axis=0, keepdims=True)
    z = (z - mu) / jnp.sqrt(var + 1e-5) * bg_ref[...] + bb_ref[...]
    z = jnp.maximum(z, 0.0)
    h2 = jnp.dot(z, w2_ref[...], preferred_element_type=jnp.float32) + b2_ref[...]
    r = jnp.dot(h2, rw_ref[...], preferred_element_type=jnp.float32) + rb_ref[...]
    ind = (lax.broadcasted_iota(jnp.int32, (G, N), 0) == batch_ref[...])
    out_ref[...] = jnp.dot(ind.astype(jnp.float32), r,
                           preferred_element_type=jnp.float32)


def _tc_call(body, out_shape):
    return pl.pallas_call(body, out_shape=jax.ShapeDtypeStruct(out_shape,
                                                               jnp.float32))


def _make_sc_agg():
    mesh = plsc.VectorSubcoreMesh(core_axis_name="c", subcore_axis_name="s",
                                  num_cores=NC, num_subcores=NS)

    @functools.partial(
        pl.kernel,
        mesh=mesh,
        out_type=jax.ShapeDtypeStruct((NC, N, H), jnp.float32),
        scratch_types=[
            pltpu.VMEM((RS, 2, K), jnp.int32),
            pltpu.VMEM((K, H), jnp.float32),
            pltpu.VMEM((K, H), jnp.float32),
            pltpu.VMEM((16, H), jnp.float32),
            pltpu.VMEM_SHARED((N, H), jnp.float32),
            pltpu.SemaphoreType.DMA,
            pltpu.SemaphoreType.DMA,
            pltpu.SemaphoreType.DMA,
            pltpu.SemaphoreType.DMA,
            pltpu.SemaphoreType.DMA,
        ],
    )
    def sc_agg(h_hbm, ei_hbm, out_hbm,
               ring, rows0, rows1, zbuf, agg_sh,
               sem0, sem1, semz, semi0, semi1):
        cid = lax.axis_index("c")
        sid = lax.axis_index("s")
        wid = cid * NS + sid
        cw = CWB + jnp.where(wid >= NW - NXW, 1, 0)
        sw = CWB * wid + jnp.maximum(wid - (NW - NXW), 0)

        zero = jnp.zeros((16,), jnp.float32)
        for r in range(16):
            for c in range(H // 16):
                zbuf[r, pl.ds(c * 16, 16)] = zero

        def zbody(j, carry):
            off = pl.multiple_of(jnp.minimum(sid * ZR + j * 16, N - 16), 16)
            pltpu.async_copy(zbuf, agg_sh.at[pl.ds(off, 16)], semz)
            return carry
        lax.fori_loop(0, ZR // 16, zbody, 0)

        def stage(t, slot, sem):
            off = pl.multiple_of((sw + t) * K, 128)
            pltpu.async_copy(ei_hbm.at[pl.ds(0, 2), pl.ds(off, K)],
                             ring.at[slot], sem)

        def stage_wait(sem):
            pltpu.make_async_copy(ei_hbm.at[pl.ds(0, 2), pl.ds(0, K)],
                                  ring.at[0], sem).wait()

        def gwait(buf, sem):
            pltpu.make_async_copy(h_hbm.at[ring.at[0, 0]], buf, sem).wait()

        stage(0, 0, semi0)
        stage_wait(semi0)
        stage(1, 1, semi1)
        stage(2, 2, semi0)
        pltpu.async_copy(h_hbm.at[ring.at[0, 0]], rows0, sem0)

        def zdrain(j, carry):
            pltpu.make_async_copy(zbuf, agg_sh.at[pl.ds(0, 16)], semz).wait()
            return carry
        lax.fori_loop(0, ZR // 16, zdrain, 0)
        plsc.subcore_barrier()

        def gbody(g, carry):
            t = 2 * g
            s1 = lax.rem(t + 1, RS)
            s2 = lax.rem(t + 2, RS)
            s3 = lax.rem(t + 3, RS)
            s0 = lax.rem(t, RS)
            stage_wait(semi1)
            pltpu.async_copy(h_hbm.at[ring.at[s1, 0]], rows1, sem1)

            @pl.when(t + 3 < cw)
            def _():
                stage(t + 3, s3, semi1)

            gwait(rows0, sem0)
            pltpu.sync_copy(rows0, agg_sh.at[ring.at[s0, 1]], add=True)

            @pl.when(t + 2 < cw)
            def _():
                stage_wait(semi0)
                pltpu.async_copy(h_hbm.at[ring.at[s2, 0]], rows0, sem0)

            @pl.when(t + 4 < cw)
            def _():
                stage(t + 4, s0, semi0)

            gwait(rows1, sem1)
            pltpu.sync_copy(rows1, agg_sh.at[ring.at[s1, 1]], add=True)
            return carry
        lax.fori_loop(0, CWB // 2, gbody, 0)

        @pl.when(cw > CWB)
        def _():
            gwait(rows0, sem0)
            pltpu.sync_copy(
                rows0, agg_sh.at[ring.at[lax.rem(CWB, RS), 1]], add=True)
        plsc.subcore_barrier()

        wb = pl.multiple_of(jnp.minimum(sid * ZR, N - ZR), 16)
        pltpu.sync_copy(agg_sh.at[pl.ds(wb, ZR)],
                        out_hbm.at[cid, pl.ds(wb, ZR)])

    return sc_agg


_SC_AGG_CACHE = []


def _sc_agg(h, ei):
    if not _SC_AGG_CACHE:
        _SC_AGG_CACHE.append(_make_sc_agg())
    return _SC_AGG_CACHE[0](h, ei)


def kernel(x, edge_index, batch, embed,
           norm0_gamma, norm0_beta, lin0_W, lin0_b, eps0,
           mlp0_W1, mlp0_b1, mlp0_bn_gamma, mlp0_bn_beta, mlp0_W2, mlp0_b2,
           norm1_gamma, norm1_beta, lin1_W, lin1_b, eps1,
           mlp1_W1, mlp1_b1, mlp1_bn_gamma, mlp1_bn_beta, mlp1_W2, mlp1_b2,
           readout_W, readout_b):
    x2 = x.reshape(N, 1)
    batch2 = batch.reshape(1, N)
    row = lambda v: v.reshape(1, -1)

    h1 = _tc_call(_embed_layer_body, (N, H))(
        x2, embed, row(norm0_gamma), row(norm0_beta), lin0_W, row(lin0_b))

    p0 = _sc_agg(h1, edge_index)

    h3 = _tc_call(_mid_body, (N, H))(
        h1, p0, eps0.reshape(1, 1), mlp0_W1, row(mlp0_b1),
        row(mlp0_bn_gamma), row(mlp0_bn_beta), mlp0_W2, row(mlp0_b2),
        row(norm1_gamma), row(norm1_beta), lin1_W, row(lin1_b))

    p1 = _sc_agg(h3, edge_index)

    out = _tc_call(_final_body, (G, OUT))(
        h3, p1, eps1.reshape(1, 1), mlp1_W1, row(mlp1_b1),
        row(mlp1_bn_gamma), row(mlp1_bn_beta), mlp1_W2, row(mlp1_b2),
        readout_W, row(readout_b), batch2)

    return out


# speedup vs baseline: 13.3193x; 1.0608x over previous
import functools

import jax
import jax.numpy as jnp
from jax import lax
from jax.experimental import pallas as pl
from jax.experimental.pallas import tpu as pltpu
from jax.experimental.pallas import tpu_sc as plsc

N = 10000
E = 320000
H = 128
OUT = 128
G = 64
NVOCAB = 100

NC = 2
NS = 16
NW = NC * NS
K = 128
NCH = E // K
CWB = NCH // NW
NXW = NCH - CWB * NW
RS = 4
ZR = 640


def _bn(x, gamma, beta):
    mu = jnp.mean(x, axis=0, keepdims=True)
    var = jnp.mean(x * x, axis=0, keepdims=True) - mu * mu
    return (x - mu) / jnp.sqrt(var + 1e-5) * gamma + beta


def _embed_layer_body(x_ref, embed_ref, ng_ref, nb_ref, lw_ref, lb_ref, out_ref):
    xi = x_ref[...]
    onehot = (lax.broadcasted_iota(jnp.int32, (NVOCAB, N), 0) == xi[None, :])
    h = lax.dot_general(onehot.astype(jnp.float32), embed_ref[...],
                        (((0,), (0,)), ((), ())),
                        preferred_element_type=jnp.float32)
    hn = _bn(h, ng_ref[...], nb_ref[...])
    out_ref[...] = jnp.maximum(
        jnp.dot(hn, lw_ref[...], preferred_element_type=jnp.float32)
        + lb_ref[...], 0.0)


def _mid_body(h_ref, p_ref, eps_ref, w1_ref, b1_ref, bg_ref, bb_ref,
              w2_ref, b2_ref, ng_ref, nb_ref, lw_ref, lb_ref, out_ref):
    h = h_ref[...]
    agg = p_ref[0] + p_ref[1]
    z = (1.0 + eps_ref[0, 0]) * h + agg
    z = jnp.dot(z, w1_ref[...], preferred_element_type=jnp.float32) + b1_ref[...]
    z = jnp.maximum(_bn(z, bg_ref[...], bb_ref[...]), 0.0)
    h2 = jnp.dot(z, w2_ref[...], preferred_element_type=jnp.float32) + b2_ref[...]
    hn = _bn(h2, ng_ref[...], nb_ref[...])
    out_ref[...] = jnp.maximum(
        jnp.dot(hn, lw_ref[...], preferred_element_type=jnp.float32)
        + lb_ref[...], 0.0)


def _final_body(h_ref, p_ref, eps_ref, w1_ref, b1_ref, bg_ref, bb_ref,
                w2_ref, b2_ref, rw_ref, rb_ref, batch_ref, out_ref):
    h = h_ref[...]
    agg = p_ref[0] + p_ref[1]
    z = (1.0 + eps_ref[0, 0]) * h + agg
    z = jnp.dot(z, w1_ref[...], preferred_element_type=jnp.float32) + b1_ref[...]
    z = jnp.maximum(_bn(z, bg_ref[...], bb_ref[...]), 0.0)
    h2 = jnp.dot(z, w2_ref[...], preferred_element_type=jnp.float32) + b2_ref[...]
    r = jnp.dot(h2, rw_ref[...], preferred_element_type=jnp.float32) + rb_ref[...]
    ind = (lax.broadcasted_iota(jnp.int32, (G, N), 0) == batch_ref[...][None, :])
    out_ref[...] = jnp.dot(ind.astype(jnp.float32), r,
                           preferred_element_type=jnp.float32)


def _tc_call(body, out_shape):
    return pl.pallas_call(body, out_shape=jax.ShapeDtypeStruct(out_shape,
                                                               jnp.float32))


def _make_sc_agg():
    mesh = plsc.VectorSubcoreMesh(core_axis_name="c", subcore_axis_name="s",
                                  num_cores=NC, num_subcores=NS)

    @functools.partial(
        pl.kernel,
        mesh=mesh,
        out_type=jax.ShapeDtypeStruct((NC, N, H), jnp.float32),
        scratch_types=[
            pltpu.VMEM((RS, 2, K), jnp.int32),
            pltpu.VMEM((K, H), jnp.float32),
            pltpu.VMEM((K, H), jnp.float32),
            pltpu.VMEM((16, H), jnp.float32),
            pltpu.VMEM_SHARED((N, H), jnp.float32),
            pltpu.SemaphoreType.DMA,
            pltpu.SemaphoreType.DMA,
            pltpu.SemaphoreType.DMA,
            pltpu.SemaphoreType.DMA,
            pltpu.SemaphoreType.DMA,
        ],
    )
    def sc_agg(h_hbm, ei_hbm, out_hbm,
               ring, rows0, rows1, zbuf, agg_sh,
               sem0, sem1, semz, semi0, semi1):
        cid = lax.axis_index("c")
        sid = lax.axis_index("s")
        wid = cid * NS + sid
        cw = CWB + jnp.where(wid >= NW - NXW, 1, 0)
        sw = CWB * wid + jnp.maximum(wid - (NW - NXW), 0)

        zero = jnp.zeros((16,), jnp.float32)
        for r in range(16):
            for c in range(H // 16):
                zbuf[r, pl.ds(c * 16, 16)] = zero

        def zbody(j, carry):
            off = pl.multiple_of(jnp.minimum(sid * ZR + j * 16, N - 16), 16)
            pltpu.async_copy(zbuf, agg_sh.at[pl.ds(off, 16)], semz)
            return carry
        lax.fori_loop(0, ZR // 16, zbody, 0)

        def stage(t, slot, sem):
            off = pl.multiple_of((sw + t) * K, 128)
            pltpu.async_copy(ei_hbm.at[pl.ds(0, 2), pl.ds(off, K)],
                             ring.at[slot], sem)

        def stage_wait(sem):
            pltpu.make_async_copy(ei_hbm.at[pl.ds(0, 2), pl.ds(0, K)],
                                  ring.at[0], sem).wait()

        def gwait(buf, sem):
            pltpu.make_async_copy(h_hbm.at[ring.at[0, 0]], buf, sem).wait()

        stage(0, 0, semi0)
        stage_wait(semi0)
        stage(1, 1, semi1)
        stage(2, 2, semi0)
        pltpu.async_copy(h_hbm.at[ring.at[0, 0]], rows0, sem0)

        def zdrain(j, carry):
            pltpu.make_async_copy(zbuf, agg_sh.at[pl.ds(0, 16)], semz).wait()
            return carry
        lax.fori_loop(0, ZR // 16, zdrain, 0)
        plsc.subcore_barrier()

        def gbody(g, carry):
            t = 2 * g
            s1 = lax.rem(t + 1, RS)
            s2 = lax.rem(t + 2, RS)
            s3 = lax.rem(t + 3, RS)
            s0 = lax.rem(t, RS)
            stage_wait(semi1)
            pltpu.async_copy(h_hbm.at[ring.at[s1, 0]], rows1, sem1)

            @pl.when(t + 3 < cw)
            def _():
                stage(t + 3, s3, semi1)

            gwait(rows0, sem0)
            pltpu.sync_copy(rows0, agg_sh.at[ring.at[s0, 1]], add=True)

            @pl.when(t + 2 < cw)
            def _():
                stage_wait(semi0)
                pltpu.async_copy(h_hbm.at[ring.at[s2, 0]], rows0, sem0)

            @pl.when(t + 4 < cw)
            def _():
                stage(t + 4, s0, semi0)

            gwait(rows1, sem1)
            pltpu.sync_copy(rows1, agg_sh.at[ring.at[s1, 1]], add=True)
            return carry
        lax.fori_loop(0, CWB // 2, gbody, 0)

        @pl.when(cw > CWB)
        def _():
            gwait(rows0, sem0)
            pltpu.sync_copy(
                rows0, agg_sh.at[ring.at[lax.rem(CWB, RS), 1]], add=True)
        plsc.subcore_barrier()

        wb = pl.multiple_of(jnp.minimum(sid * ZR, N - ZR), 16)
        pltpu.sync_copy(agg_sh.at[pl.ds(wb, ZR)],
                        out_hbm.at[cid, pl.ds(wb, ZR)])

    return sc_agg


_SC_AGG_CACHE = []


def _sc_agg(h, ei):
    if not _SC_AGG_CACHE:
        _SC_AGG_CACHE.append(_make_sc_agg())
    return _SC_AGG_CACHE[0](h, ei)


def kernel(x, edge_index, batch, embed,
           norm0_gamma, norm0_beta, lin0_W, lin0_b, eps0,
           mlp0_W1, mlp0_b1, mlp0_bn_gamma, mlp0_bn_beta, mlp0_W2, mlp0_b2,
           norm1_gamma, norm1_beta, lin1_W, lin1_b, eps1,
           mlp1_W1, mlp1_b1, mlp1_bn_gamma, mlp1_bn_beta, mlp1_W2, mlp1_b2,
           readout_W, readout_b):
    row = lambda v: v.reshape(1, -1)

    h1 = _tc_call(_embed_layer_body, (N, H))(
        x, embed, row(norm0_gamma), row(norm0_beta), lin0_W, row(lin0_b))

    p0 = _sc_agg(h1, edge_index)

    h3 = _tc_call(_mid_body, (N, H))(
        h1, p0, eps0.reshape(1, 1), mlp0_W1, row(mlp0_b1),
        row(mlp0_bn_gamma), row(mlp0_bn_beta), mlp0_W2, row(mlp0_b2),
        row(norm1_gamma), row(norm1_beta), lin1_W, row(lin1_b))

    p1 = _sc_agg(h3, edge_index)

    out = _tc_call(_final_body, (G, OUT))(
        h3, p1, eps1.reshape(1, 1), mlp1_W1, row(mlp1_b1),
        row(mlp1_bn_gamma), row(mlp1_bn_beta), mlp1_W2, row(mlp1_b2),
        readout_W, row(readout_b), batch)

    return out
